# baseline (speedup 1.0000x reference)
import functools

import jax
import jax.numpy as jnp
from jax import lax
from jax.experimental import pallas as pl
from jax.experimental.pallas import tpu as pltpu
from jax.experimental.pallas import tpu_sc as plsc

NC = 2
NS = 16
NW = NC * NS
CHUNK = 128
BN_EPS = 1e-5


def _einsum_body(x_ref, w_ref, o_ref):
    o_ref[0] = jnp.dot(x_ref[...], w_ref[0], preferred_element_type=jnp.float32)


def _tc_einsum(feat, w):
    n, f = feat.shape
    k, _, fo = w.shape
    bn = 2000
    return pl.pallas_call(
        _einsum_body,
        grid=(n // bn, k),
        in_specs=[
            pl.BlockSpec((bn, f), lambda i, j: (i, 0)),
            pl.BlockSpec((1, f, fo), lambda i, j: (j, 0, 0)),
        ],
        out_specs=pl.BlockSpec((1, bn, fo), lambda i, j: (j, i, 0)),
        out_shape=jax.ShapeDtypeStruct((k, n, fo), jnp.float32),
    )(feat, w)


def _bn_silu_body(n, p_ref, g_ref, b_ref, o_ref):
    h = p_ref[0, :n] + p_ref[1, :n]
    mu = jnp.mean(h, axis=0, keepdims=True)
    var = jnp.mean((h - mu) ** 2, axis=0, keepdims=True)
    hn = (h - mu) * lax.rsqrt(var + BN_EPS) * g_ref[0] + b_ref[0]
    o_ref[...] = hn * jax.nn.sigmoid(hn)


def _tc_bn_silu(partials, gamma, beta, n):
    f = partials.shape[2]
    return pl.pallas_call(
        functools.partial(_bn_silu_body, n),
        out_shape=jax.ShapeDtypeStruct((n, f), jnp.float32),
    )(partials, gamma.reshape(1, f), beta.reshape(1, f))


def _final_body(z_ref, wp_ref, bp_ref, gp_ref, bt_ref, p_ref, o_ref):
    zp = jnp.dot(z_ref[...], wp_ref[...], preferred_element_type=jnp.float32)
    zp = zp + bp_ref[0]
    mu = jnp.mean(zp, axis=0, keepdims=True)
    var = jnp.mean((zp - mu) ** 2, axis=0, keepdims=True)
    zp = (zp - mu) * lax.rsqrt(var + BN_EPS) * gp_ref[0] + bt_ref[0]
    zp = jnp.maximum(zp, 0.0)
    n = z_ref.shape[0]
    o_ref[...] = (p_ref[0, :n] + p_ref[1, :n]) + zp


def _tc_final(z, wp, bp, gp, bt, partials):
    n, f = z.shape
    fo = wp.shape[1]
    return pl.pallas_call(
        _final_body,
        out_shape=jax.ShapeDtypeStruct((n, fo), jnp.float32),
    )(z, wp, bp.reshape(1, fo), gp.reshape(1, fo), bt.reshape(1, fo), partials)


NPHASE = 2


def _sc_edge_pass(table, src_slab, ko_slab, dst_slab, zeros, n_nodes, n_acc,
                  nchunk):
    f = table.shape[1]
    mesh = plsc.VectorSubcoreMesh(
        core_axis_name="c", subcore_axis_name="s", num_cores=NC)
    zrows = n_acc // NS
    w = nchunk // NPHASE

    @functools.partial(
        pl.kernel,
        out_type=jax.ShapeDtypeStruct((NC, n_acc, f), jnp.float32),
        mesh=mesh,
        scratch_types=[
            pltpu.VMEM((w + 2, CHUNK), jnp.int32),
            pltpu.VMEM((w, CHUNK), jnp.int32),
            pltpu.VMEM((w, CHUNK), jnp.int32),
            pltpu.VMEM((CHUNK, f), jnp.float32),
            pltpu.VMEM((CHUNK, f), jnp.float32),
            pltpu.VMEM_SHARED((n_acc, f), jnp.float32),
            pltpu.SemaphoreType.DMA,
            pltpu.SemaphoreType.DMA,
        ],
    )
    def body(table_hbm, src_hbm, ko_hbm, dst_hbm, zeros_hbm, out_hbm,
             src_v, ko_v, dst_v, rows0_v, rows1_v, acc_sh, sem0, sem1):
        c = lax.axis_index("c")
        s = lax.axis_index("s")
        wid = c * NS + s

        pltpu.sync_copy(zeros_hbm, acc_sh.at[pl.ds(s * zrows, zrows)])
        plsc.subcore_barrier()

        zvec = jnp.zeros((16,), jnp.int32)
        for r in range(2):
            for t in range(0, CHUNK, 16):
                src_v[w + r, pl.ds(t, 16)] = zvec

        def gather(j, buf, sem):
            return pltpu.make_async_copy(table_hbm.at[src_v.at[j]], buf, sem)

        def phase(p, carry):
            base = p * w
            pltpu.sync_copy(src_hbm.at[wid, pl.ds(base, w)],
                            src_v.at[pl.ds(0, w)])
            pltpu.sync_copy(ko_hbm.at[wid, pl.ds(base, w)], ko_v)
            pltpu.sync_copy(dst_hbm.at[wid, pl.ds(base, w)], dst_v)

            def to_comb(i, cc):
                j = i // 8
                t = (i % 8) * 16
                ko = ko_v[j, pl.ds(t, 16)]
                sv = src_v[j, pl.ds(t, 16)]
                src_v[j, pl.ds(t, 16)] = ko * n_nodes + sv
                return cc
            lax.fori_loop(0, w * 8, to_comb, 0)

            gather(0, rows0_v, sem0).start()

            def chunk_body(i, cc):
                jj = 2 * i
                gather(jj + 1, rows1_v, sem1).start()
                gather(jj, rows0_v, sem0).wait()
                pltpu.sync_copy(rows0_v, acc_sh.at[dst_v.at[jj]], add=True)
                gather(jj + 2, rows0_v, sem0).start()
                gather(jj + 1, rows1_v, sem1).wait()
                pltpu.sync_copy(rows1_v, acc_sh.at[dst_v.at[jj + 1]], add=True)
                return cc
            lax.fori_loop(0, w // 2, chunk_body, 0)
            gather(w, rows0_v, sem0).wait()
            return carry

        lax.fori_loop(0, NPHASE, phase, 0)

        plsc.subcore_barrier()

        pltpu.sync_copy(acc_sh.at[pl.ds(s * zrows, zrows)],
                        out_hbm.at[c, pl.ds(s * zrows, zrows)])

    return body(table, src_slab, ko_slab, dst_slab, zeros)


def kernel(x, z, edge_index, kernel_offset, W1, gamma1, beta1, W2, Wp, bp,
           gamma_p, beta_p):
    n, f = x.shape
    e = edge_index.shape[1]
    k = W1.shape[0]

    align = NW * NPHASE * 2 * CHUNK
    per_w = (-(-e // align)) * NPHASE * 2 * CHUNK
    nchunk = per_w // CHUNK
    e_pad = per_w * NW
    n_acc = -(-(n + 1) // (NS * 8)) * NS * 8

    pad = e_pad - e
    src = jnp.concatenate([edge_index[0], jnp.zeros((pad,), jnp.int32)])
    ko = jnp.concatenate([kernel_offset, jnp.zeros((pad,), jnp.int32)])
    dst = jnp.concatenate([edge_index[1], jnp.full((pad,), n, jnp.int32)])
    src_slab = src.reshape(NW, nchunk, CHUNK)
    ko_slab = ko.reshape(NW, nchunk, CHUNK)
    dst_slab = dst.reshape(NW, nchunk, CHUNK)
    zeros = jnp.zeros((n_acc // NS, f), jnp.float32)

    y1 = _tc_einsum(x, W1).reshape(k * n, f)
    p1 = _sc_edge_pass(y1, src_slab, ko_slab, dst_slab, zeros, n, n_acc, nchunk)
    h = _tc_bn_silu(p1, gamma1, beta1, n)

    y2 = _tc_einsum(h, W2).reshape(k * n, f)
    p2 = _sc_edge_pass(y2, src_slab, ko_slab, dst_slab, zeros, n, n_acc, nchunk)

    out = _tc_final(z, Wp, bp, gamma_p, beta_p, p2)
    return (out, out)


# speedup vs baseline: 1.3116x; 1.3116x over previous
import functools

import jax
import jax.numpy as jnp
from jax import lax
from jax.experimental import pallas as pl
from jax.experimental.pallas import tpu as pltpu
from jax.experimental.pallas import tpu_sc as plsc

NC = 2
NS = 16
NW = NC * NS
CHUNK = 128
BN_EPS = 1e-5


def _einsum_body(x_ref, w_ref, o_ref):
    o_ref[0] = jnp.dot(x_ref[...], w_ref[0], preferred_element_type=jnp.float32)


def _tc_einsum(feat, w):
    n, f = feat.shape
    k, _, fo = w.shape
    bn = 2000
    return pl.pallas_call(
        _einsum_body,
        grid=(n // bn, k),
        in_specs=[
            pl.BlockSpec((bn, f), lambda i, j: (i, 0)),
            pl.BlockSpec((1, f, fo), lambda i, j: (j, 0, 0)),
        ],
        out_specs=pl.BlockSpec((1, bn, fo), lambda i, j: (j, i, 0)),
        out_shape=jax.ShapeDtypeStruct((k, n, fo), jnp.float32),
    )(feat, w)


def _bn_silu_body(n, p_ref, g_ref, b_ref, o_ref):
    h = p_ref[0, :n] + p_ref[1, :n]
    mu = jnp.mean(h, axis=0, keepdims=True)
    var = jnp.mean((h - mu) ** 2, axis=0, keepdims=True)
    hn = (h - mu) * lax.rsqrt(var + BN_EPS) * g_ref[0] + b_ref[0]
    o_ref[...] = hn * jax.nn.sigmoid(hn)


def _tc_bn_silu(partials, gamma, beta, n):
    f = partials.shape[2]
    return pl.pallas_call(
        functools.partial(_bn_silu_body, n),
        out_shape=jax.ShapeDtypeStruct((n, f), jnp.float32),
    )(partials, gamma.reshape(1, f), beta.reshape(1, f))


def _final_body(z_ref, wp_ref, bp_ref, gp_ref, bt_ref, p_ref, o_ref):
    zp = jnp.dot(z_ref[...], wp_ref[...], preferred_element_type=jnp.float32)
    zp = zp + bp_ref[0]
    mu = jnp.mean(zp, axis=0, keepdims=True)
    var = jnp.mean((zp - mu) ** 2, axis=0, keepdims=True)
    zp = (zp - mu) * lax.rsqrt(var + BN_EPS) * gp_ref[0] + bt_ref[0]
    zp = jnp.maximum(zp, 0.0)
    n = z_ref.shape[0]
    o_ref[...] = (p_ref[0, :n] + p_ref[1, :n]) + zp


def _tc_final(z, wp, bp, gp, bt, partials):
    n, f = z.shape
    fo = wp.shape[1]
    return pl.pallas_call(
        _final_body,
        out_shape=jax.ShapeDtypeStruct((n, fo), jnp.float32),
    )(z, wp, bp.reshape(1, fo), gp.reshape(1, fo), bt.reshape(1, fo), partials)


NPHASE = 2

NCH_BY_CORE = (64, 96)


def _sc_edge_pass(table, src_slab, ko_slab, dst_slab, zeros, n_nodes, n_acc):
    f = table.shape[1]
    mesh = plsc.VectorSubcoreMesh(
        core_axis_name="c", subcore_axis_name="s", num_cores=NC)
    zrows = n_acc // NS
    w_max = max(NCH_BY_CORE) // NPHASE

    @functools.partial(
        pl.kernel,
        out_type=jax.ShapeDtypeStruct((NC, n_acc, f), jnp.float32),
        mesh=mesh,
        scratch_types=[
            pltpu.VMEM((w_max, CHUNK), jnp.int32),
            pltpu.VMEM((w_max, CHUNK), jnp.int32),
            pltpu.VMEM((w_max, CHUNK), jnp.int32),
            pltpu.VMEM((CHUNK, f), jnp.float32),
            pltpu.VMEM_SHARED((n_acc, f), jnp.float32),
            pltpu.SemaphoreType.DMA,
        ],
    )
    def body(table_hbm, src_hbm, ko_hbm, dst_hbm, zeros_hbm, out_hbm,
             src_v, ko_v, dst_v, rows_v, acc_sh, sem):
        c = lax.axis_index("c")
        s = lax.axis_index("s")
        wid = c * NS + s
        w_c = jnp.where(c == 0, NCH_BY_CORE[0] // NPHASE,
                        NCH_BY_CORE[1] // NPHASE)

        pltpu.sync_copy(zeros_hbm, acc_sh.at[pl.ds(s * zrows, zrows)])
        plsc.subcore_barrier()

        def phase(p, carry):
            base = p * w_c
            pltpu.sync_copy(src_hbm.at[wid, pl.ds(base, w_max)], src_v)
            pltpu.sync_copy(ko_hbm.at[wid, pl.ds(base, w_max)], ko_v)
            pltpu.sync_copy(dst_hbm.at[wid, pl.ds(base, w_max)], dst_v)

            def to_comb(i, cc):
                j = i // 8
                t = (i % 8) * 16
                ko = ko_v[j, pl.ds(t, 16)]
                sv = src_v[j, pl.ds(t, 16)]
                src_v[j, pl.ds(t, 16)] = ko * n_nodes + sv
                return cc
            lax.fori_loop(0, w_max * 8, to_comb, 0)

            def chunk_body(j, cc):
                pltpu.async_copy(table_hbm.at[src_v.at[j]], rows_v, sem).wait()
                pltpu.sync_copy(rows_v, acc_sh.at[dst_v.at[j]], add=True)
                return cc
            lax.fori_loop(0, w_c, chunk_body, 0)
            return carry

        lax.fori_loop(0, NPHASE, phase, 0)

        plsc.subcore_barrier()

        pltpu.sync_copy(acc_sh.at[pl.ds(s * zrows, zrows)],
                        out_hbm.at[c, pl.ds(s * zrows, zrows)])

    return body(table, src_slab, ko_slab, dst_slab, zeros)


def kernel(x, z, edge_index, kernel_offset, W1, gamma1, beta1, W2, Wp, bp,
           gamma_p, beta_p):
    n, f = x.shape
    e = edge_index.shape[1]
    k = W1.shape[0]

    nch0, nch1 = NCH_BY_CORE
    assert NS * (nch0 + nch1) * CHUNK >= e
    nch_max = max(nch0, nch1)
    e0 = NS * nch0 * CHUNK
    e1 = NS * nch1 * CHUNK
    n_acc = -(-(n + 1) // (NS * 8)) * NS * 8

    def mk_slab(arr, fill):
        a = jnp.concatenate(
            [arr, jnp.full((e0 + e1 - e,), fill, jnp.int32)])
        parts = []
        for lo, hi, nch in ((0, e0, nch0), (e0, e0 + e1, nch1)):
            p = a[lo:hi].reshape(NS, nch, CHUNK)
            if nch < nch_max:
                p = jnp.concatenate(
                    [p, jnp.full((NS, nch_max - nch, CHUNK), fill,
                                 jnp.int32)], axis=1)
            parts.append(p)
        return jnp.concatenate(parts, axis=0)

    src_slab = mk_slab(edge_index[0], 0)
    ko_slab = mk_slab(kernel_offset, 0)
    dst_slab = mk_slab(edge_index[1], n)
    zeros = jnp.zeros((n_acc // NS, f), jnp.float32)

    y1 = _tc_einsum(x, W1).reshape(k * n, f)
    p1 = _sc_edge_pass(y1, src_slab, ko_slab, dst_slab, zeros, n, n_acc)
    h = _tc_bn_silu(p1, gamma1, beta1, n)

    y2 = _tc_einsum(h, W2).reshape(k * n, f)
    p2 = _sc_edge_pass(y2, src_slab, ko_slab, dst_slab, zeros, n, n_acc)

    out = _tc_final(z, Wp, bp, gamma_p, beta_p, p2)
    return (out, out)


# speedup vs baseline: 1.3130x; 1.0011x over previous
import functools

import jax
import jax.numpy as jnp
from jax import lax
from jax.experimental import pallas as pl
from jax.experimental.pallas import tpu as pltpu
from jax.experimental.pallas import tpu_sc as plsc

NC = 2
NS = 16
NW = NC * NS
CHUNK = 128
BN_EPS = 1e-5


def _einsum_body(x_ref, w_ref, o_ref):
    o_ref[0] = jnp.dot(x_ref[...], w_ref[0], preferred_element_type=jnp.float32)


def _tc_einsum(feat, w):
    n, f = feat.shape
    k, _, fo = w.shape
    bn = 2000
    return pl.pallas_call(
        _einsum_body,
        grid=(n // bn, k),
        in_specs=[
            pl.BlockSpec((bn, f), lambda i, j: (i, 0)),
            pl.BlockSpec((1, f, fo), lambda i, j: (j, 0, 0)),
        ],
        out_specs=pl.BlockSpec((1, bn, fo), lambda i, j: (j, i, 0)),
        out_shape=jax.ShapeDtypeStruct((k, n, fo), jnp.float32),
    )(feat, w)


def _bn_silu_body(n, p_ref, g_ref, b_ref, o_ref):
    h = p_ref[0, :n] + p_ref[1, :n]
    mu = jnp.mean(h, axis=0, keepdims=True)
    var = jnp.mean((h - mu) ** 2, axis=0, keepdims=True)
    hn = (h - mu) * lax.rsqrt(var + BN_EPS) * g_ref[0] + b_ref[0]
    o_ref[...] = hn * jax.nn.sigmoid(hn)


def _tc_bn_silu(partials, gamma, beta, n):
    f = partials.shape[2]
    return pl.pallas_call(
        functools.partial(_bn_silu_body, n),
        out_shape=jax.ShapeDtypeStruct((n, f), jnp.float32),
    )(partials, gamma.reshape(1, f), beta.reshape(1, f))


def _final_body(z_ref, wp_ref, bp_ref, gp_ref, bt_ref, p_ref, o_ref):
    zp = jnp.dot(z_ref[...], wp_ref[...], preferred_element_type=jnp.float32)
    zp = zp + bp_ref[0]
    mu = jnp.mean(zp, axis=0, keepdims=True)
    var = jnp.mean((zp - mu) ** 2, axis=0, keepdims=True)
    zp = (zp - mu) * lax.rsqrt(var + BN_EPS) * gp_ref[0] + bt_ref[0]
    zp = jnp.maximum(zp, 0.0)
    n = z_ref.shape[0]
    o_ref[...] = (p_ref[0, :n] + p_ref[1, :n]) + zp


def _tc_final(z, wp, bp, gp, bt, partials):
    n, f = z.shape
    fo = wp.shape[1]
    return pl.pallas_call(
        _final_body,
        out_shape=jax.ShapeDtypeStruct((n, fo), jnp.float32),
    )(z, wp, bp.reshape(1, fo), gp.reshape(1, fo), bt.reshape(1, fo), partials)


NPHASE = 2

NCH_BY_CORE = (96, 64)


def _sc_edge_pass(table, src_slab, ko_slab, dst_slab, zeros, n_nodes, n_acc):
    f = table.shape[1]
    mesh = plsc.VectorSubcoreMesh(
        core_axis_name="c", subcore_axis_name="s", num_cores=NC)
    zrows = n_acc // NS
    w_max = max(NCH_BY_CORE) // NPHASE

    @functools.partial(
        pl.kernel,
        out_type=jax.ShapeDtypeStruct((NC, n_acc, f), jnp.float32),
        mesh=mesh,
        scratch_types=[
            pltpu.VMEM((w_max, CHUNK), jnp.int32),
            pltpu.VMEM((w_max, CHUNK), jnp.int32),
            pltpu.VMEM((w_max, CHUNK), jnp.int32),
            pltpu.VMEM((CHUNK, f), jnp.float32),
            pltpu.VMEM_SHARED((n_acc, f), jnp.float32),
            pltpu.SemaphoreType.DMA,
        ],
    )
    def body(table_hbm, src_hbm, ko_hbm, dst_hbm, zeros_hbm, out_hbm,
             src_v, ko_v, dst_v, rows_v, acc_sh, sem):
        c = lax.axis_index("c")
        s = lax.axis_index("s")
        wid = c * NS + s
        w_c = jnp.where(c == 0, NCH_BY_CORE[0] // NPHASE,
                        NCH_BY_CORE[1] // NPHASE)

        pltpu.sync_copy(zeros_hbm, acc_sh.at[pl.ds(s * zrows, zrows)])
        plsc.subcore_barrier()

        def phase(p, carry):
            base = p * w_c
            pltpu.sync_copy(src_hbm.at[wid, pl.ds(base, w_max)], src_v)
            pltpu.sync_copy(ko_hbm.at[wid, pl.ds(base, w_max)], ko_v)
            pltpu.sync_copy(dst_hbm.at[wid, pl.ds(base, w_max)], dst_v)

            def to_comb(i, cc):
                j = i // 8
                t = (i % 8) * 16
                ko = ko_v[j, pl.ds(t, 16)]
                sv = src_v[j, pl.ds(t, 16)]
                src_v[j, pl.ds(t, 16)] = ko * n_nodes + sv
                return cc
            lax.fori_loop(0, w_max * 8, to_comb, 0)

            def chunk_body(j, cc):
                pltpu.async_copy(table_hbm.at[src_v.at[j]], rows_v, sem).wait()
                pltpu.sync_copy(rows_v, acc_sh.at[dst_v.at[j]], add=True)
                return cc
            lax.fori_loop(0, w_c, chunk_body, 0)
            return carry

        lax.fori_loop(0, NPHASE, phase, 0)

        plsc.subcore_barrier()

        pltpu.sync_copy(acc_sh.at[pl.ds(s * zrows, zrows)],
                        out_hbm.at[c, pl.ds(s * zrows, zrows)])

    return body(table, src_slab, ko_slab, dst_slab, zeros)


def kernel(x, z, edge_index, kernel_offset, W1, gamma1, beta1, W2, Wp, bp,
           gamma_p, beta_p):
    n, f = x.shape
    e = edge_index.shape[1]
    k = W1.shape[0]

    nch0, nch1 = NCH_BY_CORE
    assert NS * (nch0 + nch1) * CHUNK >= e
    nch_max = max(nch0, nch1)
    e0 = NS * nch0 * CHUNK
    e1 = NS * nch1 * CHUNK
    n_acc = -(-(n + 1) // (NS * 8)) * NS * 8

    def mk_slab(arr, fill):
        a = jnp.concatenate(
            [arr, jnp.full((e0 + e1 - e,), fill, jnp.int32)])
        parts = []
        for lo, hi, nch in ((0, e0, nch0), (e0, e0 + e1, nch1)):
            p = a[lo:hi].reshape(NS, nch, CHUNK)
            if nch < nch_max:
                p = jnp.concatenate(
                    [p, jnp.full((NS, nch_max - nch, CHUNK), fill,
                                 jnp.int32)], axis=1)
            parts.append(p)
        return jnp.concatenate(parts, axis=0)

    src_slab = mk_slab(edge_index[0], 0)
    ko_slab = mk_slab(kernel_offset, 0)
    dst_slab = mk_slab(edge_index[1], n)
    zeros = jnp.zeros((n_acc // NS, f), jnp.float32)

    y1 = _tc_einsum(x, W1).reshape(k * n, f)
    p1 = _sc_edge_pass(y1, src_slab, ko_slab, dst_slab, zeros, n, n_acc)
    h = _tc_bn_silu(p1, gamma1, beta1, n)

    y2 = _tc_einsum(h, W2).reshape(k * n, f)
    p2 = _sc_edge_pass(y2, src_slab, ko_slab, dst_slab, zeros, n, n_acc)

    out = _tc_final(z, Wp, bp, gamma_p, beta_p, p2)
    return (out, out)
